# baseline (speedup 1.0000x reference)
import functools

import jax
import jax.numpy as jnp
from jax import lax
from jax.experimental import pallas as pl
from jax.experimental.pallas import tpu as pltpu
from jax.experimental.pallas import tpu_sc as plsc

NC = 2
NS = 16
NW = NC * NS
K = 80


SPMEM_BUDGET = 2097151 * 4


@functools.lru_cache(maxsize=None)
def _make_sc_scatter(n, e, d, with_cnt):
    assert e % (NW * K) == 0 and n % K == 0 and d % 16 == 0
    epw = e // NW
    nch = epw // K
    nrc = n // K

    cnt_bytes = (n * 64 + NS * 2 * 64 * K) if with_cnt else 0
    nbuf = 2
    for cand in (3, 4):
        if n * d * 4 + NS * (cand * K * d * 4 + 2 * cand * 2 * K * 4) \
                + cnt_bytes < SPMEM_BUDGET:
            nbuf = cand
    ng = 2
    ns = nbuf - ng
    ir = 2 * nbuf
    assert nch > ir and ns >= 1

    mesh = plsc.VectorSubcoreMesh(core_axis_name="c", subcore_axis_name="s")

    out_type = [jax.ShapeDtypeStruct((NC, n, d), jnp.float32)]
    cnt_scratch = []
    if with_cnt:
        out_type.append(jax.ShapeDtypeStruct((NC, n, 16), jnp.float32))
        cnt_scratch = [
            pltpu.VMEM((K, 16), jnp.float32),
            pltpu.VMEM((K, 16), jnp.float32),
            pltpu.VMEM_SHARED((n, 16), jnp.float32),
        ]

    @functools.partial(
        pl.kernel,
        out_type=tuple(out_type) if with_cnt else out_type[0],
        mesh=mesh,
        scratch_types=[
            *[pltpu.VMEM((2, K), jnp.int32) for _ in range(ir)],
            *[pltpu.VMEM((K, d), jnp.float32) for _ in range(nbuf)],
            pltpu.VMEM_SHARED((n, d), jnp.float32),
            *cnt_scratch,
            *[pltpu.SemaphoreType.DMA
              for _ in range(ir + nbuf + nbuf + (nbuf if with_cnt else 0))],
        ],
        compiler_params=pltpu.CompilerParams(use_tc_tiling_on_sc=False),
    )
    def sc_kernel(*args):
        it = iter(args)
        rows_hbm = next(it)
        edges_hbm = next(it)
        part_hbm = next(it)
        pcnt_hbm = next(it) if with_cnt else None
        ix = [next(it) for _ in range(ir)]
        rows_v = [next(it) for _ in range(nbuf)]
        acc = next(it)
        if with_cnt:
            ones_v, zcol_v, acc1 = next(it), next(it), next(it)
        isem = [next(it) for _ in range(ir)]
        gsem = [next(it) for _ in range(nbuf)]
        ssem = [next(it) for _ in range(nbuf)]
        csem = [next(it) for _ in range(nbuf)] if with_cnt else None

        c = lax.axis_index("c")
        s = lax.axis_index("s")
        wid = s * NC + c
        ch0 = wid * nch

        def ixload(i, a):
            return pltpu.make_async_copy(edges_hbm.at[ch0 + i], ix[a], isem[a])

        def gather(b, a):
            return pltpu.make_async_copy(
                rows_hbm.at[ix[a].at[0]], rows_v[b], gsem[b])

        def scatter_start(b, a):
            pltpu.async_copy(rows_v[b], acc.at[ix[a].at[1]], ssem[b],
                             add=True)
            if with_cnt:
                pltpu.async_copy(ones_v, acc1.at[ix[a].at[1]], csem[b],
                                 add=True)

        def scatter_wait(b, a):
            pltpu.make_async_copy(rows_v[b], acc.at[ix[a].at[1]],
                                  ssem[b]).wait()
            if with_cnt:
                pltpu.make_async_copy(ones_v, acc1.at[ix[a].at[1]],
                                      csem[b]).wait()

        def zrow(i, carry):
            for j in range(d // 16):
                rows_v[0][i, pl.ds(j * 16, 16)] = jnp.zeros((16,), jnp.float32)
            if with_cnt:
                ones_v[i, pl.ds(0, 16)] = jnp.ones((16,), jnp.float32)
                zcol_v[i, pl.ds(0, 16)] = jnp.zeros((16,), jnp.float32)
            return carry
        lax.fori_loop(0, K, zrow, 0)

        for a in range(ir):
            ixload(a, a).start()

        def zacc(i, carry):
            t = s + i * NS
            @pl.when(t < nrc)
            def _():
                pltpu.sync_copy(rows_v[0], acc.at[pl.ds(t * K, K)])
                if with_cnt:
                    pltpu.sync_copy(zcol_v, acc1.at[pl.ds(t * K, K)])
            return carry
        lax.fori_loop(0, (nrc + NS - 1) // NS, zacc, 0)
        plsc.subcore_barrier()

        for q in range(ng):
            ixload(q, q).wait()
            gather(q, q).start()

        def step(i, q, tail):
            b, a = q % nbuf, q % ir
            gather(b, a).wait()
            scatter_start(b, a)

            def retire():
                scatter_wait((q + ng) % nbuf, (q - ns) % ir)
            def refill():
                ixload(i + ir - ns, (q - ns) % ir).start()
            def advance():
                a3 = (q + ng) % ir
                ixload(i + ng, a3).wait()
                gather((q + ng) % nbuf, a3).start()
            if tail:
                if i >= ns:
                    retire()
                if i >= ns and i + ir - ns < nch:
                    refill()
                if i + ng < nch:
                    advance()
            else:
                pl.when(i >= ns)(retire)
                pl.when(jnp.logical_and(i >= ns, i + ir - ns < nch))(refill)
                pl.when(i + ng < nch)(advance)

        def outer(j, carry):
            for q in range(ir):
                step(j * ir + q, q, False)
            return carry
        lax.fori_loop(0, nch // ir, outer, 0)
        for i in range((nch // ir) * ir, nch):
            step(i, i % ir, True)
        for i in range(nch - ns, nch):
            scatter_wait(i % nbuf, i % ir)
        plsc.subcore_barrier()

        def cout(i, carry):
            t = s + i * NS
            @pl.when(t < nrc)
            def _():
                pltpu.sync_copy(acc.at[pl.ds(t * K, K)],
                                part_hbm.at[c, pl.ds(t * K, K)])
                if with_cnt:
                    pltpu.sync_copy(acc1.at[pl.ds(t * K, K)],
                                    pcnt_hbm.at[c, pl.ds(t * K, K)])
            return carry
        lax.fori_loop(0, (nrc + NS - 1) // NS, cout, 0)

    return sc_kernel


@functools.lru_cache(maxsize=None)
def _make_tc1(n, f_in, hid, f_out, r):
    grid = n // r

    def body(part, pcnt, x, wl1, bl1, wr1, wl2, wr2, bl2, p, r2, inv):
        a = part[0] + part[1]
        cnt = pcnt[0, :, 0:1] + pcnt[1, :, 0:1]
        iv = 1.0 / jnp.maximum(cnt, 1.0)
        mean = a * iv
        h = jnp.maximum(
            jnp.dot(mean, wl1[...], preferred_element_type=jnp.float32)
            + bl1[...]
            + jnp.dot(x[...], wr1[...], preferred_element_type=jnp.float32),
            0.0)
        p[...] = jnp.dot(h, wl2[...], preferred_element_type=jnp.float32)
        r2[...] = (jnp.dot(h, wr2[...], preferred_element_type=jnp.float32)
                   + bl2[...])
        inv[...] = iv

    return pl.pallas_call(
        body,
        grid=(grid,),
        in_specs=[
            pl.BlockSpec((NC, r, f_in), lambda i: (0, i, 0)),
            pl.BlockSpec((NC, r, 16), lambda i: (0, i, 0)),
            pl.BlockSpec((r, f_in), lambda i: (i, 0)),
            pl.BlockSpec((f_in, hid), lambda i: (0, 0)),
            pl.BlockSpec((1, hid), lambda i: (0, 0)),
            pl.BlockSpec((f_in, hid), lambda i: (0, 0)),
            pl.BlockSpec((hid, f_out), lambda i: (0, 0)),
            pl.BlockSpec((hid, f_out), lambda i: (0, 0)),
            pl.BlockSpec((1, f_out), lambda i: (0, 0)),
        ],
        out_specs=[
            pl.BlockSpec((r, f_out), lambda i: (i, 0)),
            pl.BlockSpec((r, f_out), lambda i: (i, 0)),
            pl.BlockSpec((r, 1), lambda i: (i, 0)),
        ],
        out_shape=[
            jax.ShapeDtypeStruct((n, f_out), jnp.float32),
            jax.ShapeDtypeStruct((n, f_out), jnp.float32),
            jax.ShapeDtypeStruct((n, 1), jnp.float32),
        ],
    )


@functools.lru_cache(maxsize=None)
def _make_tc2(n, f_out, r):
    grid = n // r

    def body(part, inv, r2, out):
        out[...] = (part[0] + part[1]) * inv[...] + r2[...]

    return pl.pallas_call(
        body,
        grid=(grid,),
        in_specs=[
            pl.BlockSpec((NC, r, f_out), lambda i: (0, i, 0)),
            pl.BlockSpec((r, 1), lambda i: (i, 0)),
            pl.BlockSpec((r, f_out), lambda i: (i, 0)),
        ],
        out_specs=pl.BlockSpec((r, f_out), lambda i: (i, 0)),
        out_shape=jax.ShapeDtypeStruct((n, f_out), jnp.float32),
    )


def kernel(x, edge_index, W_l1, b_l1, W_r1, W_l2, b_l2, W_r2):
    n, f_in = x.shape
    e = edge_index.shape[1]
    hid = W_l1.shape[1]
    f_out = W_l2.shape[1]

    edges = edge_index.reshape(2, e // K, K).transpose(1, 0, 2)

    part1, pcnt = _make_sc_scatter(n, e, f_in, True)(x, edges)
    p, r2, inv = _make_tc1(n, f_in, hid, f_out, 400)(
        part1, pcnt, x, W_l1, b_l1.reshape(1, hid), W_r1, W_l2, W_r2,
        b_l2.reshape(1, f_out))
    part2 = _make_sc_scatter(n, e, f_out, False)(p, edges)
    out = _make_tc2(n, f_out, 400)(part2, inv, r2)
    return out


# speedup vs baseline: 1.0432x; 1.0432x over previous
import functools

import jax
import jax.numpy as jnp
from jax import lax
from jax.experimental import pallas as pl
from jax.experimental.pallas import tpu as pltpu
from jax.experimental.pallas import tpu_sc as plsc

NC = 2
NS = 16
NW = NC * NS
K = 80


SPMEM_BUDGET = 2097151 * 4


@functools.lru_cache(maxsize=None)
def _make_sc_scatter(n, e, d, with_cnt):
    assert e % (NW * K) == 0 and n % K == 0 and d % 16 == 0
    epw = e // NW
    nch = epw // K
    nrc = n // K

    cnt_bytes = (n * 64 + NS * 2 * 64 * K) if with_cnt else 0
    nbuf = 2
    for cand in (3, 4):
        if n * d * 4 + NS * (cand * K * d * 4 + 2 * cand * 2 * K * 4) \
                + cnt_bytes < SPMEM_BUDGET:
            nbuf = cand
    ng = nbuf - 1
    ns = nbuf - ng
    ir = 2 * nbuf
    assert nch > ir and ns >= 1

    mesh = plsc.VectorSubcoreMesh(core_axis_name="c", subcore_axis_name="s")

    out_type = [jax.ShapeDtypeStruct((NC, n, d), jnp.float32)]
    cnt_scratch = []
    if with_cnt:
        out_type.append(jax.ShapeDtypeStruct((NC, n, 16), jnp.float32))
        cnt_scratch = [
            pltpu.VMEM((K, 16), jnp.float32),
            pltpu.VMEM((K, 16), jnp.float32),
            pltpu.VMEM_SHARED((n, 16), jnp.float32),
        ]

    @functools.partial(
        pl.kernel,
        out_type=tuple(out_type) if with_cnt else out_type[0],
        mesh=mesh,
        scratch_types=[
            *[pltpu.VMEM((2, K), jnp.int32) for _ in range(ir)],
            *[pltpu.VMEM((K, d), jnp.float32) for _ in range(nbuf)],
            pltpu.VMEM_SHARED((n, d), jnp.float32),
            *cnt_scratch,
            *[pltpu.SemaphoreType.DMA
              for _ in range(ir + nbuf + nbuf + (nbuf if with_cnt else 0))],
        ],
        compiler_params=pltpu.CompilerParams(use_tc_tiling_on_sc=False),
    )
    def sc_kernel(*args):
        it = iter(args)
        rows_hbm = next(it)
        edges_hbm = next(it)
        part_hbm = next(it)
        pcnt_hbm = next(it) if with_cnt else None
        ix = [next(it) for _ in range(ir)]
        rows_v = [next(it) for _ in range(nbuf)]
        acc = next(it)
        if with_cnt:
            ones_v, zcol_v, acc1 = next(it), next(it), next(it)
        isem = [next(it) for _ in range(ir)]
        gsem = [next(it) for _ in range(nbuf)]
        ssem = [next(it) for _ in range(nbuf)]
        csem = [next(it) for _ in range(nbuf)] if with_cnt else None

        c = lax.axis_index("c")
        s = lax.axis_index("s")
        wid = s * NC + c
        ch0 = wid * nch

        def ixload(i, a):
            return pltpu.make_async_copy(edges_hbm.at[ch0 + i], ix[a], isem[a])

        def gather(b, a):
            return pltpu.make_async_copy(
                rows_hbm.at[ix[a].at[0]], rows_v[b], gsem[b])

        def scatter_start(b, a):
            pltpu.async_copy(rows_v[b], acc.at[ix[a].at[1]], ssem[b],
                             add=True)
            if with_cnt:
                pltpu.async_copy(ones_v, acc1.at[ix[a].at[1]], csem[b],
                                 add=True)

        def scatter_wait(b, a):
            pltpu.make_async_copy(rows_v[b], acc.at[ix[a].at[1]],
                                  ssem[b]).wait()
            if with_cnt:
                pltpu.make_async_copy(ones_v, acc1.at[ix[a].at[1]],
                                      csem[b]).wait()

        def zrow(i, carry):
            for j in range(d // 16):
                rows_v[0][i, pl.ds(j * 16, 16)] = jnp.zeros((16,), jnp.float32)
            if with_cnt:
                ones_v[i, pl.ds(0, 16)] = jnp.ones((16,), jnp.float32)
                zcol_v[i, pl.ds(0, 16)] = jnp.zeros((16,), jnp.float32)
            return carry
        lax.fori_loop(0, K, zrow, 0)

        for a in range(ir):
            ixload(a, a).start()

        def zacc(i, carry):
            t = s + i * NS
            @pl.when(t < nrc)
            def _():
                pltpu.sync_copy(rows_v[0], acc.at[pl.ds(t * K, K)])
                if with_cnt:
                    pltpu.sync_copy(zcol_v, acc1.at[pl.ds(t * K, K)])
            return carry
        lax.fori_loop(0, (nrc + NS - 1) // NS, zacc, 0)
        plsc.subcore_barrier()

        for q in range(ng):
            ixload(q, q).wait()
            gather(q, q).start()

        def step(i, q, tail):
            b, a = q % nbuf, q % ir
            gather(b, a).wait()
            scatter_start(b, a)

            def retire():
                scatter_wait((q + ng) % nbuf, (q - ns) % ir)
            def refill():
                ixload(i + ir - ns, (q - ns) % ir).start()
            def advance():
                a3 = (q + ng) % ir
                ixload(i + ng, a3).wait()
                gather((q + ng) % nbuf, a3).start()
            if tail:
                if i >= ns:
                    retire()
                if i >= ns and i + ir - ns < nch:
                    refill()
                if i + ng < nch:
                    advance()
            else:
                pl.when(i >= ns)(retire)
                pl.when(jnp.logical_and(i >= ns, i + ir - ns < nch))(refill)
                pl.when(i + ng < nch)(advance)

        def outer(j, carry):
            for q in range(ir):
                step(j * ir + q, q, False)
            return carry
        lax.fori_loop(0, nch // ir, outer, 0)
        for i in range((nch // ir) * ir, nch):
            step(i, i % ir, True)
        for i in range(nch - ns, nch):
            scatter_wait(i % nbuf, i % ir)
        plsc.subcore_barrier()

        def cout(i, carry):
            t = s + i * NS
            @pl.when(t < nrc)
            def _():
                pltpu.sync_copy(acc.at[pl.ds(t * K, K)],
                                part_hbm.at[c, pl.ds(t * K, K)])
                if with_cnt:
                    pltpu.sync_copy(acc1.at[pl.ds(t * K, K)],
                                    pcnt_hbm.at[c, pl.ds(t * K, K)])
            return carry
        lax.fori_loop(0, (nrc + NS - 1) // NS, cout, 0)

    return sc_kernel


@functools.lru_cache(maxsize=None)
def _make_tc1(n, f_in, hid, f_out, r):
    grid = n // r

    def body(part, pcnt, x, wl1, bl1, wr1, wl2, wr2, bl2, p, r2, inv):
        a = part[0] + part[1]
        cnt = pcnt[0, :, 0:1] + pcnt[1, :, 0:1]
        iv = 1.0 / jnp.maximum(cnt, 1.0)
        mean = a * iv
        h = jnp.maximum(
            jnp.dot(mean, wl1[...], preferred_element_type=jnp.float32)
            + bl1[...]
            + jnp.dot(x[...], wr1[...], preferred_element_type=jnp.float32),
            0.0)
        p[...] = jnp.dot(h, wl2[...], preferred_element_type=jnp.float32)
        r2[...] = (jnp.dot(h, wr2[...], preferred_element_type=jnp.float32)
                   + bl2[...])
        inv[...] = iv

    return pl.pallas_call(
        body,
        grid=(grid,),
        in_specs=[
            pl.BlockSpec((NC, r, f_in), lambda i: (0, i, 0)),
            pl.BlockSpec((NC, r, 16), lambda i: (0, i, 0)),
            pl.BlockSpec((r, f_in), lambda i: (i, 0)),
            pl.BlockSpec((f_in, hid), lambda i: (0, 0)),
            pl.BlockSpec((1, hid), lambda i: (0, 0)),
            pl.BlockSpec((f_in, hid), lambda i: (0, 0)),
            pl.BlockSpec((hid, f_out), lambda i: (0, 0)),
            pl.BlockSpec((hid, f_out), lambda i: (0, 0)),
            pl.BlockSpec((1, f_out), lambda i: (0, 0)),
        ],
        out_specs=[
            pl.BlockSpec((r, f_out), lambda i: (i, 0)),
            pl.BlockSpec((r, f_out), lambda i: (i, 0)),
            pl.BlockSpec((r, 1), lambda i: (i, 0)),
        ],
        out_shape=[
            jax.ShapeDtypeStruct((n, f_out), jnp.float32),
            jax.ShapeDtypeStruct((n, f_out), jnp.float32),
            jax.ShapeDtypeStruct((n, 1), jnp.float32),
        ],
    )


@functools.lru_cache(maxsize=None)
def _make_tc2(n, f_out, r):
    grid = n // r

    def body(part, inv, r2, out):
        out[...] = (part[0] + part[1]) * inv[...] + r2[...]

    return pl.pallas_call(
        body,
        grid=(grid,),
        in_specs=[
            pl.BlockSpec((NC, r, f_out), lambda i: (0, i, 0)),
            pl.BlockSpec((r, 1), lambda i: (i, 0)),
            pl.BlockSpec((r, f_out), lambda i: (i, 0)),
        ],
        out_specs=pl.BlockSpec((r, f_out), lambda i: (i, 0)),
        out_shape=jax.ShapeDtypeStruct((n, f_out), jnp.float32),
    )


def kernel(x, edge_index, W_l1, b_l1, W_r1, W_l2, b_l2, W_r2):
    n, f_in = x.shape
    e = edge_index.shape[1]
    hid = W_l1.shape[1]
    f_out = W_l2.shape[1]

    edges = edge_index.reshape(2, e // K, K).transpose(1, 0, 2)

    part1, pcnt = _make_sc_scatter(n, e, f_in, True)(x, edges)
    p, r2, inv = _make_tc1(n, f_in, hid, f_out, 400)(
        part1, pcnt, x, W_l1, b_l1.reshape(1, hid), W_r1, W_l2, W_r2,
        b_l2.reshape(1, f_out))
    part2 = _make_sc_scatter(n, e, f_out, False)(p, edges)
    out = _make_tc2(n, f_out, 400)(part2, inv, r2)
    return out


# speedup vs baseline: 1.0901x; 1.0450x over previous
import functools

import jax
import jax.numpy as jnp
from jax import lax
from jax.experimental import pallas as pl
from jax.experimental.pallas import tpu as pltpu
from jax.experimental.pallas import tpu_sc as plsc

NC = 2
NS = 16
NW = NC * NS
K = 80


SPMEM_BUDGET = 2097151 * 4


@functools.lru_cache(maxsize=None)
def _make_sc_scatter(n, e, d, with_cnt):
    assert e % (NW * K) == 0 and n % K == 0 and d % 16 == 0
    epw = e // NW
    nch = epw // K
    nrc = n // K

    cnt_bytes = (n * 64 + NS * 2 * 64 * K) if with_cnt else 0
    nbuf = 2
    for cand in (3, 4):
        if n * d * 4 + NS * (cand * K * d * 4 + 2 * cand * 2 * K * 4) \
                + cnt_bytes < SPMEM_BUDGET:
            nbuf = cand
    ir = 2 * nbuf
    assert nch > ir

    mesh = plsc.VectorSubcoreMesh(core_axis_name="c", subcore_axis_name="s")

    out_type = [jax.ShapeDtypeStruct((NC, n, d), jnp.float32)]
    cnt_scratch = []
    if with_cnt:
        out_type.append(jax.ShapeDtypeStruct((NC, n, 16), jnp.float32))
        cnt_scratch = [
            pltpu.VMEM((K, 16), jnp.float32),
            pltpu.VMEM((K, 16), jnp.float32),
            pltpu.VMEM_SHARED((n, 16), jnp.float32),
        ]

    @functools.partial(
        pl.kernel,
        out_type=tuple(out_type) if with_cnt else out_type[0],
        mesh=mesh,
        scratch_types=[
            *[pltpu.VMEM((2, K), jnp.int32) for _ in range(ir)],
            *[pltpu.VMEM((K, d), jnp.float32) for _ in range(nbuf)],
            pltpu.VMEM_SHARED((n, d), jnp.float32),
            *cnt_scratch,
            *[pltpu.SemaphoreType.DMA for _ in range(ir + nbuf)],
        ],
        compiler_params=pltpu.CompilerParams(use_tc_tiling_on_sc=False),
    )
    def sc_kernel(*args):
        it = iter(args)
        rows_hbm = next(it)
        edges_hbm = next(it)
        part_hbm = next(it)
        pcnt_hbm = next(it) if with_cnt else None
        ix = [next(it) for _ in range(ir)]
        rows_v = [next(it) for _ in range(nbuf)]
        acc = next(it)
        if with_cnt:
            ones_v, zcol_v, acc1 = next(it), next(it), next(it)
        isem = [next(it) for _ in range(ir)]
        gsem = [next(it) for _ in range(nbuf)]

        c = lax.axis_index("c")
        s = lax.axis_index("s")
        wid = s * NC + c
        ch0 = wid * nch

        def ixload(i, a):
            return pltpu.make_async_copy(edges_hbm.at[ch0 + i], ix[a], isem[a])

        def gather(b, a):
            return pltpu.make_async_copy(
                rows_hbm.at[ix[a].at[0]], rows_v[b], gsem[b])

        def scatter(b, a):
            pltpu.sync_copy(rows_v[b], acc.at[ix[a].at[1]], add=True)
            if with_cnt:
                pltpu.sync_copy(ones_v, acc1.at[ix[a].at[1]], add=True)

        def zrow(i, carry):
            for j in range(d // 16):
                rows_v[0][i, pl.ds(j * 16, 16)] = jnp.zeros((16,), jnp.float32)
            if with_cnt:
                ones_v[i, pl.ds(0, 16)] = jnp.ones((16,), jnp.float32)
                zcol_v[i, pl.ds(0, 16)] = jnp.zeros((16,), jnp.float32)
            return carry
        lax.fori_loop(0, K, zrow, 0)

        for a in range(ir):
            ixload(a, a).start()

        def zacc(i, carry):
            t = s + i * NS
            @pl.when(t < nrc)
            def _():
                pltpu.sync_copy(rows_v[0], acc.at[pl.ds(t * K, K)])
                if with_cnt:
                    pltpu.sync_copy(zcol_v, acc1.at[pl.ds(t * K, K)])
            return carry
        lax.fori_loop(0, (nrc + NS - 1) // NS, zacc, 0)
        plsc.subcore_barrier()

        for q in range(nbuf):
            ixload(q, q).wait()
            gather(q, q).start()

        def step(i, q, tail):
            b, a = q % nbuf, q % ir
            gather(b, a).wait()
            scatter(b, a)

            def refill():
                ixload(i + ir, a).start()
            def advance():
                a2 = (q + nbuf) % ir
                ixload(i + nbuf, a2).wait()
                gather(b, a2).start()
            if tail:
                if i + ir < nch:
                    refill()
                if i + nbuf < nch:
                    advance()
            else:
                pl.when(i + ir < nch)(refill)
                pl.when(i + nbuf < nch)(advance)

        def outer(j, carry):
            for q in range(ir):
                step(j * ir + q, q, False)
            return carry
        lax.fori_loop(0, nch // ir, outer, 0)
        for i in range((nch // ir) * ir, nch):
            step(i, i % ir, True)
        plsc.subcore_barrier()

        def cout(i, carry):
            t = s + i * NS
            @pl.when(t < nrc)
            def _():
                pltpu.sync_copy(acc.at[pl.ds(t * K, K)],
                                part_hbm.at[c, pl.ds(t * K, K)])
                if with_cnt:
                    pltpu.sync_copy(acc1.at[pl.ds(t * K, K)],
                                    pcnt_hbm.at[c, pl.ds(t * K, K)])
            return carry
        lax.fori_loop(0, (nrc + NS - 1) // NS, cout, 0)

    return sc_kernel


@functools.lru_cache(maxsize=None)
def _make_tc1(n, f_in, hid, f_out, r):
    grid = n // r

    def body(part, pcnt, x, wl1, bl1, wr1, wl2, wr2, bl2, p, r2, inv):
        a = part[0] + part[1]
        cnt = pcnt[0, :, 0:1] + pcnt[1, :, 0:1]
        iv = 1.0 / jnp.maximum(cnt, 1.0)
        mean = a * iv
        h = jnp.maximum(
            jnp.dot(mean, wl1[...], preferred_element_type=jnp.float32)
            + bl1[...]
            + jnp.dot(x[...], wr1[...], preferred_element_type=jnp.float32),
            0.0)
        p[...] = jnp.dot(h, wl2[...], preferred_element_type=jnp.float32)
        r2[...] = (jnp.dot(h, wr2[...], preferred_element_type=jnp.float32)
                   + bl2[...])
        inv[...] = iv

    return pl.pallas_call(
        body,
        grid=(grid,),
        in_specs=[
            pl.BlockSpec((NC, r, f_in), lambda i: (0, i, 0)),
            pl.BlockSpec((NC, r, 16), lambda i: (0, i, 0)),
            pl.BlockSpec((r, f_in), lambda i: (i, 0)),
            pl.BlockSpec((f_in, hid), lambda i: (0, 0)),
            pl.BlockSpec((1, hid), lambda i: (0, 0)),
            pl.BlockSpec((f_in, hid), lambda i: (0, 0)),
            pl.BlockSpec((hid, f_out), lambda i: (0, 0)),
            pl.BlockSpec((hid, f_out), lambda i: (0, 0)),
            pl.BlockSpec((1, f_out), lambda i: (0, 0)),
        ],
        out_specs=[
            pl.BlockSpec((r, f_out), lambda i: (i, 0)),
            pl.BlockSpec((r, f_out), lambda i: (i, 0)),
            pl.BlockSpec((r, 1), lambda i: (i, 0)),
        ],
        out_shape=[
            jax.ShapeDtypeStruct((n, f_out), jnp.float32),
            jax.ShapeDtypeStruct((n, f_out), jnp.float32),
            jax.ShapeDtypeStruct((n, 1), jnp.float32),
        ],
    )


@functools.lru_cache(maxsize=None)
def _make_tc2(n, f_out, r):
    grid = n // r

    def body(part, inv, r2, out):
        out[...] = (part[0] + part[1]) * inv[...] + r2[...]

    return pl.pallas_call(
        body,
        grid=(grid,),
        in_specs=[
            pl.BlockSpec((NC, r, f_out), lambda i: (0, i, 0)),
            pl.BlockSpec((r, 1), lambda i: (i, 0)),
            pl.BlockSpec((r, f_out), lambda i: (i, 0)),
        ],
        out_specs=pl.BlockSpec((r, f_out), lambda i: (i, 0)),
        out_shape=jax.ShapeDtypeStruct((n, f_out), jnp.float32),
    )


def kernel(x, edge_index, W_l1, b_l1, W_r1, W_l2, b_l2, W_r2):
    n, f_in = x.shape
    e = edge_index.shape[1]
    hid = W_l1.shape[1]
    f_out = W_l2.shape[1]

    edges = edge_index.reshape(2, e // K, K).transpose(1, 0, 2)

    part1, pcnt = _make_sc_scatter(n, e, f_in, True)(x, edges)
    p, r2, inv = _make_tc1(n, f_in, hid, f_out, 400)(
        part1, pcnt, x, W_l1, b_l1.reshape(1, hid), W_r1, W_l2, W_r2,
        b_l2.reshape(1, f_out))
    part2 = _make_sc_scatter(n, e, f_out, False)(p, edges)
    out = _make_tc2(n, f_out, 400)(part2, inv, r2)
    return out


# speedup vs baseline: 1.1036x; 1.0124x over previous
import functools

import jax
import jax.numpy as jnp
from jax import lax
from jax.experimental import pallas as pl
from jax.experimental.pallas import tpu as pltpu
from jax.experimental.pallas import tpu_sc as plsc

NC = 2
NS = 16
NW = NC * NS
K = 80


SPMEM_BUDGET = 2097151 * 4


@functools.lru_cache(maxsize=None)
def _make_sc_scatter(n, e, d, with_cnt):
    assert e % (NW * K) == 0 and n % K == 0 and d % 16 == 0
    epw = e // NW
    nch = epw // K
    nrc = n // K

    cnt_bytes = (n * 32 + NS * 2 * 32 * K) if with_cnt else 0
    nbuf = 2
    for cand in (3, 4):
        if n * d * 4 + NS * (cand * K * d * 4 + 2 * cand * 2 * K * 4) \
                + cnt_bytes < SPMEM_BUDGET:
            nbuf = cand
    ir = 2 * nbuf
    assert nch > ir

    mesh = plsc.VectorSubcoreMesh(core_axis_name="c", subcore_axis_name="s")

    out_type = [jax.ShapeDtypeStruct((NC, n, d), jnp.float32)]
    cnt_scratch = []
    if with_cnt:
        out_type.append(jax.ShapeDtypeStruct((NC, n, 16), jnp.bfloat16))
        cnt_scratch = [
            pltpu.VMEM((K, 16), jnp.bfloat16),
            pltpu.VMEM((K, 16), jnp.bfloat16),
            pltpu.VMEM_SHARED((n, 16), jnp.bfloat16),
        ]

    @functools.partial(
        pl.kernel,
        out_type=tuple(out_type) if with_cnt else out_type[0],
        mesh=mesh,
        scratch_types=[
            *[pltpu.VMEM((2, K), jnp.int32) for _ in range(ir)],
            *[pltpu.VMEM((K, d), jnp.float32) for _ in range(nbuf)],
            pltpu.VMEM_SHARED((n, d), jnp.float32),
            *cnt_scratch,
            *[pltpu.SemaphoreType.DMA for _ in range(ir + nbuf)],
        ],
        compiler_params=pltpu.CompilerParams(use_tc_tiling_on_sc=False),
    )
    def sc_kernel(*args):
        it = iter(args)
        rows_hbm = next(it)
        edges_hbm = next(it)
        part_hbm = next(it)
        pcnt_hbm = next(it) if with_cnt else None
        ix = [next(it) for _ in range(ir)]
        rows_v = [next(it) for _ in range(nbuf)]
        acc = next(it)
        if with_cnt:
            ones_v, zcol_v, acc1 = next(it), next(it), next(it)
        isem = [next(it) for _ in range(ir)]
        gsem = [next(it) for _ in range(nbuf)]

        c = lax.axis_index("c")
        s = lax.axis_index("s")
        wid = s * NC + c
        ch0 = wid * nch

        def ixload(i, a):
            return pltpu.make_async_copy(edges_hbm.at[ch0 + i], ix[a], isem[a])

        def gather(b, a):
            return pltpu.make_async_copy(
                rows_hbm.at[ix[a].at[0]], rows_v[b], gsem[b])

        def scatter(b, a):
            pltpu.sync_copy(rows_v[b], acc.at[ix[a].at[1]], add=True)
            if with_cnt:
                pltpu.sync_copy(ones_v, acc1.at[ix[a].at[1]], add=True)

        def zrow(i, carry):
            for j in range(d // 16):
                rows_v[0][i, pl.ds(j * 16, 16)] = jnp.zeros((16,), jnp.float32)
            return carry
        lax.fori_loop(0, K, zrow, 0)
        if with_cnt:
            def zcnt(i, carry):
                ones_v[pl.ds(2 * i, 2), :] = jnp.ones((2, 16), jnp.bfloat16)
                zcol_v[pl.ds(2 * i, 2), :] = jnp.zeros((2, 16), jnp.bfloat16)
                return carry
            lax.fori_loop(0, K // 2, zcnt, 0)

        for a in range(ir):
            ixload(a, a).start()

        def zacc(i, carry):
            t = s + i * NS
            @pl.when(t < nrc)
            def _():
                pltpu.sync_copy(rows_v[0], acc.at[pl.ds(t * K, K)])
                if with_cnt:
                    pltpu.sync_copy(zcol_v, acc1.at[pl.ds(t * K, K)])
            return carry
        lax.fori_loop(0, (nrc + NS - 1) // NS, zacc, 0)
        plsc.subcore_barrier()

        for q in range(nbuf):
            ixload(q, q).wait()
            gather(q, q).start()

        def step(i, q, tail):
            b, a = q % nbuf, q % ir
            gather(b, a).wait()
            scatter(b, a)

            def refill():
                ixload(i + ir, a).start()
            def advance():
                a2 = (q + nbuf) % ir
                ixload(i + nbuf, a2).wait()
                gather(b, a2).start()
            if tail:
                if i + ir < nch:
                    refill()
                if i + nbuf < nch:
                    advance()
            else:
                pl.when(i + ir < nch)(refill)
                pl.when(i + nbuf < nch)(advance)

        def outer(j, carry):
            for q in range(ir):
                step(j * ir + q, q, False)
            return carry
        lax.fori_loop(0, nch // ir, outer, 0)
        for i in range((nch // ir) * ir, nch):
            step(i, i % ir, True)
        plsc.subcore_barrier()

        def cout(i, carry):
            t = s + i * NS
            @pl.when(t < nrc)
            def _():
                pltpu.sync_copy(acc.at[pl.ds(t * K, K)],
                                part_hbm.at[c, pl.ds(t * K, K)])
                if with_cnt:
                    pltpu.sync_copy(acc1.at[pl.ds(t * K, K)],
                                    pcnt_hbm.at[c, pl.ds(t * K, K)])
            return carry
        lax.fori_loop(0, (nrc + NS - 1) // NS, cout, 0)

    return sc_kernel


@functools.lru_cache(maxsize=None)
def _make_tc1(n, f_in, hid, f_out, r):
    grid = n // r

    def body(part, pcnt, x, wl1, bl1, wr1, wl2, wr2, bl2, p, r2, inv):
        a = part[0] + part[1]
        cnt = (pcnt[0, :, 0:1].astype(jnp.float32)
               + pcnt[1, :, 0:1].astype(jnp.float32))
        iv = 1.0 / jnp.maximum(cnt, 1.0)
        mean = a * iv
        h = jnp.maximum(
            jnp.dot(mean, wl1[...], preferred_element_type=jnp.float32)
            + bl1[...]
            + jnp.dot(x[...], wr1[...], preferred_element_type=jnp.float32),
            0.0)
        p[...] = jnp.dot(h, wl2[...], preferred_element_type=jnp.float32)
        r2[...] = (jnp.dot(h, wr2[...], preferred_element_type=jnp.float32)
                   + bl2[...])
        inv[...] = iv

    return pl.pallas_call(
        body,
        grid=(grid,),
        in_specs=[
            pl.BlockSpec((NC, r, f_in), lambda i: (0, i, 0)),
            pl.BlockSpec((NC, r, 16), lambda i: (0, i, 0)),
            pl.BlockSpec((r, f_in), lambda i: (i, 0)),
            pl.BlockSpec((f_in, hid), lambda i: (0, 0)),
            pl.BlockSpec((1, hid), lambda i: (0, 0)),
            pl.BlockSpec((f_in, hid), lambda i: (0, 0)),
            pl.BlockSpec((hid, f_out), lambda i: (0, 0)),
            pl.BlockSpec((hid, f_out), lambda i: (0, 0)),
            pl.BlockSpec((1, f_out), lambda i: (0, 0)),
        ],
        out_specs=[
            pl.BlockSpec((r, f_out), lambda i: (i, 0)),
            pl.BlockSpec((r, f_out), lambda i: (i, 0)),
            pl.BlockSpec((r, 1), lambda i: (i, 0)),
        ],
        out_shape=[
            jax.ShapeDtypeStruct((n, f_out), jnp.float32),
            jax.ShapeDtypeStruct((n, f_out), jnp.float32),
            jax.ShapeDtypeStruct((n, 1), jnp.float32),
        ],
    )


@functools.lru_cache(maxsize=None)
def _make_tc2(n, f_out, r):
    grid = n // r

    def body(part, inv, r2, out):
        out[...] = (part[0] + part[1]) * inv[...] + r2[...]

    return pl.pallas_call(
        body,
        grid=(grid,),
        in_specs=[
            pl.BlockSpec((NC, r, f_out), lambda i: (0, i, 0)),
            pl.BlockSpec((r, 1), lambda i: (i, 0)),
            pl.BlockSpec((r, f_out), lambda i: (i, 0)),
        ],
        out_specs=pl.BlockSpec((r, f_out), lambda i: (i, 0)),
        out_shape=jax.ShapeDtypeStruct((n, f_out), jnp.float32),
    )


def kernel(x, edge_index, W_l1, b_l1, W_r1, W_l2, b_l2, W_r2):
    n, f_in = x.shape
    e = edge_index.shape[1]
    hid = W_l1.shape[1]
    f_out = W_l2.shape[1]

    edges = edge_index.reshape(2, e // K, K).transpose(1, 0, 2)

    part1, pcnt = _make_sc_scatter(n, e, f_in, True)(x, edges)
    p, r2, inv = _make_tc1(n, f_in, hid, f_out, 400)(
        part1, pcnt, x, W_l1, b_l1.reshape(1, hid), W_r1, W_l2, W_r2,
        b_l2.reshape(1, f_out))
    part2 = _make_sc_scatter(n, e, f_out, False)(p, edges)
    out = _make_tc2(n, f_out, 400)(part2, inv, r2)
    return out


# speedup vs baseline: 1.1743x; 1.0640x over previous
import functools

import jax
import jax.numpy as jnp
from jax import lax
from jax.experimental import pallas as pl
from jax.experimental.pallas import tpu as pltpu
from jax.experimental.pallas import tpu_sc as plsc

NC = 2
NS = 16
NW = NC * NS
K = 80


SPMEM_BUDGET = 2097151 * 4


@functools.lru_cache(maxsize=None)
def _make_sc_scatter(n, e, d, with_cnt):
    assert e % (NW * K) == 0 and n % K == 0 and d % 16 == 0
    epw = e // NW
    nch = epw // K
    nrc = n // K

    cnt_bytes = (n * 32 + NS * 2 * 32 * K) if with_cnt else 0
    nbuf = 2
    for cand in (3, 4):
        if n * d * 4 + NS * (cand * K * d * 4 + 2 * cand * 2 * K * 4) \
                + cnt_bytes < SPMEM_BUDGET:
            nbuf = cand
    ir = 2 * nbuf
    assert nch > ir

    mesh = plsc.VectorSubcoreMesh(core_axis_name="c", subcore_axis_name="s")

    out_type = [jax.ShapeDtypeStruct((NC, n, d), jnp.float32)]
    cnt_scratch = []
    if with_cnt:
        out_type.append(jax.ShapeDtypeStruct((NC, n, 16), jnp.bfloat16))
        cnt_scratch = [
            pltpu.VMEM((K, 16), jnp.bfloat16),
            pltpu.VMEM((K, 16), jnp.bfloat16),
            pltpu.VMEM_SHARED((n, 16), jnp.bfloat16),
        ]

    @functools.partial(
        pl.kernel,
        out_type=tuple(out_type) if with_cnt else out_type[0],
        mesh=mesh,
        scratch_types=[
            *[pltpu.VMEM((2, K), jnp.int32) for _ in range(ir)],
            *[pltpu.VMEM((K, d), jnp.float32) for _ in range(nbuf)],
            pltpu.VMEM_SHARED((n, d), jnp.float32),
            *cnt_scratch,
            *[pltpu.SemaphoreType.DMA for _ in range(ir + nbuf)],
        ],
        compiler_params=pltpu.CompilerParams(use_tc_tiling_on_sc=False),
    )
    def sc_kernel(*args):
        it = iter(args)
        rows_hbm = next(it)
        edges_hbm = next(it)
        part_hbm = next(it)
        pcnt_hbm = next(it) if with_cnt else None
        ix = [next(it) for _ in range(ir)]
        rows_v = [next(it) for _ in range(nbuf)]
        acc = next(it)
        if with_cnt:
            ones_v, zcol_v, acc1 = next(it), next(it), next(it)
        isem = [next(it) for _ in range(ir)]
        gsem = [next(it) for _ in range(nbuf)]

        c = lax.axis_index("c")
        s = lax.axis_index("s")
        wid = s * NC + c
        ch0 = wid * nch

        def ixload(i, a):
            return pltpu.make_async_copy(edges_hbm.at[ch0 + i], ix[a], isem[a])

        def gather(b, a):
            return pltpu.make_async_copy(
                rows_hbm.at[ix[a].at[0]], rows_v[b], gsem[b])

        def scatter(b, a):
            pltpu.sync_copy(rows_v[b], acc.at[ix[a].at[1]], add=True)
            if with_cnt:
                pltpu.sync_copy(ones_v, acc1.at[ix[a].at[1]], add=True)

        def zrow(i, carry):
            for j in range(d // 16):
                rows_v[0][i, pl.ds(j * 16, 16)] = jnp.zeros((16,), jnp.float32)
            return carry
        lax.fori_loop(0, K, zrow, 0)
        if with_cnt:
            def zcnt(i, carry):
                ones_v[pl.ds(2 * i, 2), :] = jnp.ones((2, 16), jnp.bfloat16)
                zcol_v[pl.ds(2 * i, 2), :] = jnp.zeros((2, 16), jnp.bfloat16)
                return carry
            lax.fori_loop(0, K // 2, zcnt, 0)

        for a in range(ir):
            ixload(a, a).start()

        def zacc(i, carry):
            t = s + i * NS
            @pl.when(t < nrc)
            def _():
                pltpu.sync_copy(rows_v[0], acc.at[pl.ds(t * K, K)])
                if with_cnt:
                    pltpu.sync_copy(zcol_v, acc1.at[pl.ds(t * K, K)])
            return carry
        lax.fori_loop(0, (nrc + NS - 1) // NS, zacc, 0)
        plsc.subcore_barrier()

        for q in range(nbuf):
            ixload(q, q).wait()
            gather(q, q).start()

        def step(i, q, tail):
            b, a = q % nbuf, q % ir
            gather(b, a).wait()
            scatter(b, a)

            def refill():
                ixload(i + ir, a).start()
            def advance():
                a2 = (q + nbuf) % ir
                ixload(i + nbuf, a2).wait()
                gather(b, a2).start()
            if tail:
                if i + ir < nch:
                    refill()
                if i + nbuf < nch:
                    advance()
            else:
                pl.when(i + ir < nch)(refill)
                pl.when(i + nbuf < nch)(advance)

        def outer(j, carry):
            for q in range(ir):
                step(j * ir + q, q, False)
            return carry
        lax.fori_loop(0, nch // ir, outer, 0)
        for i in range((nch // ir) * ir, nch):
            step(i, i % ir, True)
        plsc.subcore_barrier()

        def cout(i, carry):
            t = s + i * NS
            @pl.when(t < nrc)
            def _():
                pltpu.sync_copy(acc.at[pl.ds(t * K, K)],
                                part_hbm.at[c, pl.ds(t * K, K)])
                if with_cnt:
                    pltpu.sync_copy(acc1.at[pl.ds(t * K, K)],
                                    pcnt_hbm.at[c, pl.ds(t * K, K)])
            return carry
        lax.fori_loop(0, (nrc + NS - 1) // NS, cout, 0)

    return sc_kernel


@functools.lru_cache(maxsize=None)
def _make_tc1(n, f_in, hid, f_out, r):
    grid = n // r

    def body(part, pcnt, x, wl1, bl1, wr1, wl2, wr2, bl2, p, r2, inv):
        a = part[0] + part[1]
        cnt = (pcnt[0, :, 0:1].astype(jnp.float32)
               + pcnt[1, :, 0:1].astype(jnp.float32))
        iv = 1.0 / jnp.maximum(cnt, 1.0)
        mean = a * iv
        h = jnp.maximum(
            jnp.dot(mean, wl1[...], preferred_element_type=jnp.float32)
            + bl1[...]
            + jnp.dot(x[...], wr1[...], preferred_element_type=jnp.float32),
            0.0)
        p[...] = jnp.dot(h, wl2[...], preferred_element_type=jnp.float32)
        r2[...] = (jnp.dot(h, wr2[...], preferred_element_type=jnp.float32)
                   + bl2[...])
        inv[...] = iv

    return pl.pallas_call(
        body,
        grid=(grid,),
        in_specs=[
            pl.BlockSpec((NC, r, f_in), lambda i: (0, i, 0)),
            pl.BlockSpec((NC, r, 16), lambda i: (0, i, 0)),
            pl.BlockSpec((r, f_in), lambda i: (i, 0)),
            pl.BlockSpec((f_in, hid), lambda i: (0, 0)),
            pl.BlockSpec((1, hid), lambda i: (0, 0)),
            pl.BlockSpec((f_in, hid), lambda i: (0, 0)),
            pl.BlockSpec((hid, f_out), lambda i: (0, 0)),
            pl.BlockSpec((hid, f_out), lambda i: (0, 0)),
            pl.BlockSpec((1, f_out), lambda i: (0, 0)),
        ],
        out_specs=[
            pl.BlockSpec((r, f_out), lambda i: (i, 0)),
            pl.BlockSpec((r, f_out), lambda i: (i, 0)),
            pl.BlockSpec((r, 1), lambda i: (i, 0)),
        ],
        out_shape=[
            jax.ShapeDtypeStruct((n, f_out), jnp.float32),
            jax.ShapeDtypeStruct((n, f_out), jnp.float32),
            jax.ShapeDtypeStruct((n, 1), jnp.float32),
        ],
    )


@functools.lru_cache(maxsize=None)
def _make_tc2(n, f_out, r):
    grid = n // r

    def body(part, inv, r2, out):
        out[...] = (part[0] + part[1]) * inv[...] + r2[...]

    return pl.pallas_call(
        body,
        grid=(grid,),
        in_specs=[
            pl.BlockSpec((NC, r, f_out), lambda i: (0, i, 0)),
            pl.BlockSpec((r, 1), lambda i: (i, 0)),
            pl.BlockSpec((r, f_out), lambda i: (i, 0)),
        ],
        out_specs=pl.BlockSpec((r, f_out), lambda i: (i, 0)),
        out_shape=jax.ShapeDtypeStruct((n, f_out), jnp.float32),
    )


def kernel(x, edge_index, W_l1, b_l1, W_r1, W_l2, b_l2, W_r2):
    n, f_in = x.shape
    e = edge_index.shape[1]
    hid = W_l1.shape[1]
    f_out = W_l2.shape[1]

    edges = edge_index.reshape(2, e // K, K).transpose(1, 0, 2)

    part1, pcnt = _make_sc_scatter(n, e, f_in, True)(x, edges)
    p, r2, inv = _make_tc1(n, f_in, hid, f_out, 1000)(
        part1, pcnt, x, W_l1, b_l1.reshape(1, hid), W_r1, W_l2, W_r2,
        b_l2.reshape(1, f_out))
    part2 = _make_sc_scatter(n, e, f_out, False)(p, edges)
    out = _make_tc2(n, f_out, 1000)(part2, inv, r2)
    return out


# speedup vs baseline: 1.1945x; 1.0172x over previous
import functools

import jax
import jax.numpy as jnp
from jax import lax
from jax.experimental import pallas as pl
from jax.experimental.pallas import tpu as pltpu
from jax.experimental.pallas import tpu_sc as plsc

NC = 2
NS = 16
NW = NC * NS
K = 80


SPMEM_BUDGET = 2097151 * 4


@functools.lru_cache(maxsize=None)
def _make_sc_scatter(n, e, d, with_cnt):
    assert e % (NW * K) == 0 and n % K == 0 and d % 16 == 0
    epw = e // NW
    nch = epw // K
    nrc = n // K

    cnt_bytes = (n * 32 + NS * 2 * 32 * K) if with_cnt else 0
    nbuf = 2
    for cand in (3, 4):
        if n * d * 4 + NS * (cand * K * d * 4 + 2 * cand * 2 * K * 4) \
                + cnt_bytes < SPMEM_BUDGET:
            nbuf = cand
    ir = 2 * nbuf
    assert nch > ir

    mesh = plsc.VectorSubcoreMesh(core_axis_name="c", subcore_axis_name="s")

    out_type = [jax.ShapeDtypeStruct((NC, n, d), jnp.float32)]
    cnt_scratch = []
    if with_cnt:
        out_type.append(jax.ShapeDtypeStruct((NC, n, 16), jnp.bfloat16))
        cnt_scratch = [
            pltpu.VMEM((K, 16), jnp.bfloat16),
            pltpu.VMEM((K, 16), jnp.bfloat16),
            pltpu.VMEM_SHARED((n, 16), jnp.bfloat16),
        ]

    @functools.partial(
        pl.kernel,
        out_type=tuple(out_type) if with_cnt else out_type[0],
        mesh=mesh,
        scratch_types=[
            *[pltpu.VMEM((2, K), jnp.int32) for _ in range(ir)],
            *[pltpu.VMEM((K, d), jnp.float32) for _ in range(nbuf)],
            pltpu.VMEM_SHARED((n, d), jnp.float32),
            *cnt_scratch,
            *[pltpu.SemaphoreType.DMA for _ in range(ir + nbuf)],
        ],
        compiler_params=pltpu.CompilerParams(use_tc_tiling_on_sc=False),
    )
    def sc_kernel(*args):
        it = iter(args)
        rows_hbm = next(it)
        edges_hbm = next(it)
        part_hbm = next(it)
        pcnt_hbm = next(it) if with_cnt else None
        ix = [next(it) for _ in range(ir)]
        rows_v = [next(it) for _ in range(nbuf)]
        acc = next(it)
        if with_cnt:
            ones_v, zcol_v, acc1 = next(it), next(it), next(it)
        isem = [next(it) for _ in range(ir)]
        gsem = [next(it) for _ in range(nbuf)]

        c = lax.axis_index("c")
        s = lax.axis_index("s")
        wid = s * NC + c
        ch0 = wid * nch

        def ixload(i, a):
            return pltpu.make_async_copy(edges_hbm.at[ch0 + i], ix[a], isem[a])

        def gather(b, a):
            return pltpu.make_async_copy(
                rows_hbm.at[ix[a].at[0]], rows_v[b], gsem[b])

        def scatter(b, a):
            pltpu.sync_copy(rows_v[b], acc.at[ix[a].at[1]], add=True)
            if with_cnt:
                pltpu.sync_copy(ones_v, acc1.at[ix[a].at[1]], add=True)

        def zrow(i, carry):
            for j in range(d // 16):
                rows_v[0][i, pl.ds(j * 16, 16)] = jnp.zeros((16,), jnp.float32)
            return carry
        lax.fori_loop(0, K, zrow, 0)
        if with_cnt:
            def zcnt(i, carry):
                ones_v[pl.ds(2 * i, 2), :] = jnp.ones((2, 16), jnp.bfloat16)
                zcol_v[pl.ds(2 * i, 2), :] = jnp.zeros((2, 16), jnp.bfloat16)
                return carry
            lax.fori_loop(0, K // 2, zcnt, 0)

        for a in range(ir):
            ixload(a, a).start()

        def zacc(i, carry):
            t = s + i * NS
            @pl.when(t < nrc)
            def _():
                pltpu.sync_copy(rows_v[0], acc.at[pl.ds(t * K, K)])
                if with_cnt:
                    pltpu.sync_copy(zcol_v, acc1.at[pl.ds(t * K, K)])
            return carry
        lax.fori_loop(0, (nrc + NS - 1) // NS, zacc, 0)
        plsc.subcore_barrier()

        for q in range(nbuf):
            ixload(q, q).wait()
            gather(q, q).start()

        def step(i, q, tail):
            b, a = q % nbuf, q % ir
            gather(b, a).wait()
            scatter(b, a)

            def refill():
                ixload(i + ir, a).start()
            def advance():
                a2 = (q + nbuf) % ir
                ixload(i + nbuf, a2).wait()
                gather(b, a2).start()
            if tail:
                if i + ir < nch:
                    refill()
                if i + nbuf < nch:
                    advance()
            else:
                pl.when(i + ir < nch)(refill)
                pl.when(i + nbuf < nch)(advance)

        def outer(j, carry):
            for q in range(ir):
                step(j * ir + q, q, False)
            return carry
        lax.fori_loop(0, nch // ir, outer, 0)
        for i in range((nch // ir) * ir, nch):
            step(i, i % ir, True)
        plsc.subcore_barrier()

        def cout(i, carry):
            t = s + i * NS
            @pl.when(t < nrc)
            def _():
                pltpu.sync_copy(acc.at[pl.ds(t * K, K)],
                                part_hbm.at[c, pl.ds(t * K, K)])
                if with_cnt:
                    pltpu.sync_copy(acc1.at[pl.ds(t * K, K)],
                                    pcnt_hbm.at[c, pl.ds(t * K, K)])
            return carry
        lax.fori_loop(0, (nrc + NS - 1) // NS, cout, 0)

    return sc_kernel


@functools.lru_cache(maxsize=None)
def _make_tc1(n, f_in, hid, f_out, r):
    grid = n // r

    def body(part, pcnt, x, wl1, bl1, wr1, wl2, wr2, bl2, p, r2, inv):
        a = part[0] + part[1]
        cnt = (pcnt[0, :, 0:1].astype(jnp.float32)
               + pcnt[1, :, 0:1].astype(jnp.float32))
        iv = 1.0 / jnp.maximum(cnt, 1.0)
        mean = a * iv
        h = jnp.maximum(
            jnp.dot(mean, wl1[...], preferred_element_type=jnp.float32)
            + bl1[...]
            + jnp.dot(x[...], wr1[...], preferred_element_type=jnp.float32),
            0.0)
        p[...] = jnp.dot(h, wl2[...], preferred_element_type=jnp.float32)
        r2[...] = (jnp.dot(h, wr2[...], preferred_element_type=jnp.float32)
                   + bl2[...])
        inv[...] = iv

    return pl.pallas_call(
        body,
        grid=(grid,),
        in_specs=[
            pl.BlockSpec((NC, r, f_in), lambda i: (0, i, 0)),
            pl.BlockSpec((NC, r, 16), lambda i: (0, i, 0)),
            pl.BlockSpec((r, f_in), lambda i: (i, 0)),
            pl.BlockSpec((f_in, hid), lambda i: (0, 0)),
            pl.BlockSpec((1, hid), lambda i: (0, 0)),
            pl.BlockSpec((f_in, hid), lambda i: (0, 0)),
            pl.BlockSpec((hid, f_out), lambda i: (0, 0)),
            pl.BlockSpec((hid, f_out), lambda i: (0, 0)),
            pl.BlockSpec((1, f_out), lambda i: (0, 0)),
        ],
        out_specs=[
            pl.BlockSpec((r, f_out), lambda i: (i, 0)),
            pl.BlockSpec((r, f_out), lambda i: (i, 0)),
            pl.BlockSpec((r, 1), lambda i: (i, 0)),
        ],
        out_shape=[
            jax.ShapeDtypeStruct((n, f_out), jnp.float32),
            jax.ShapeDtypeStruct((n, f_out), jnp.float32),
            jax.ShapeDtypeStruct((n, 1), jnp.float32),
        ],
    )


@functools.lru_cache(maxsize=None)
def _make_tc2(n, f_out, r):
    grid = n // r

    def body(part, inv, r2, out):
        out[...] = (part[0] + part[1]) * inv[...] + r2[...]

    return pl.pallas_call(
        body,
        grid=(grid,),
        in_specs=[
            pl.BlockSpec((NC, r, f_out), lambda i: (0, i, 0)),
            pl.BlockSpec((r, 1), lambda i: (i, 0)),
            pl.BlockSpec((r, f_out), lambda i: (i, 0)),
        ],
        out_specs=pl.BlockSpec((r, f_out), lambda i: (i, 0)),
        out_shape=jax.ShapeDtypeStruct((n, f_out), jnp.float32),
    )


def kernel(x, edge_index, W_l1, b_l1, W_r1, W_l2, b_l2, W_r2):
    n, f_in = x.shape
    e = edge_index.shape[1]
    hid = W_l1.shape[1]
    f_out = W_l2.shape[1]

    edges = edge_index.reshape(2, e // K, K).transpose(1, 0, 2)

    part1, pcnt = _make_sc_scatter(n, e, f_in, True)(x, edges)
    p, r2, inv = _make_tc1(n, f_in, hid, f_out, 2000)(
        part1, pcnt, x, W_l1, b_l1.reshape(1, hid), W_r1, W_l2, W_r2,
        b_l2.reshape(1, f_out))
    part2 = _make_sc_scatter(n, e, f_out, False)(p, edges)
    out = _make_tc2(n, f_out, 2000)(part2, inv, r2)
    return out


# speedup vs baseline: 1.1971x; 1.0021x over previous
import functools

import jax
import jax.numpy as jnp
from jax import lax
from jax.experimental import pallas as pl
from jax.experimental.pallas import tpu as pltpu
from jax.experimental.pallas import tpu_sc as plsc

NC = 2
NS = 16
NW = NC * NS
K = 80


SPMEM_BUDGET = 2097151 * 4


@functools.lru_cache(maxsize=None)
def _make_sc_scatter(n, e, d, with_cnt):
    assert e % (NW * K) == 0 and n % K == 0 and d % 16 == 0
    epw = e // NW
    nch = epw // K
    nrc = n // K

    cnt_bytes = (n * 32 + NS * 2 * 32 * K) if with_cnt else 0
    nbuf = 2
    for cand in (3, 4):
        if n * d * 4 + NS * (cand * K * d * 4 + 2 * cand * 2 * K * 4) \
                + cnt_bytes < SPMEM_BUDGET:
            nbuf = cand
    ir = 2 * nbuf
    assert nch > ir

    mesh = plsc.VectorSubcoreMesh(core_axis_name="c", subcore_axis_name="s")

    out_type = [jax.ShapeDtypeStruct((NC, n, d), jnp.float32)]
    cnt_scratch = []
    if with_cnt:
        out_type.append(jax.ShapeDtypeStruct((NC, n, 16), jnp.bfloat16))
        cnt_scratch = [
            pltpu.VMEM((K, 16), jnp.bfloat16),
            pltpu.VMEM((K, 16), jnp.bfloat16),
            pltpu.VMEM_SHARED((n, 16), jnp.bfloat16),
        ]

    @functools.partial(
        pl.kernel,
        out_type=tuple(out_type) if with_cnt else out_type[0],
        mesh=mesh,
        scratch_types=[
            *[pltpu.VMEM((2, K), jnp.int32) for _ in range(ir)],
            *[pltpu.VMEM((K, d), jnp.float32) for _ in range(nbuf)],
            pltpu.VMEM_SHARED((n, d), jnp.float32),
            *cnt_scratch,
            *[pltpu.SemaphoreType.DMA for _ in range(ir + nbuf)],
        ],
        compiler_params=pltpu.CompilerParams(use_tc_tiling_on_sc=False),
    )
    def sc_kernel(*args):
        it = iter(args)
        rows_hbm = next(it)
        edges_hbm = next(it)
        part_hbm = next(it)
        pcnt_hbm = next(it) if with_cnt else None
        ix = [next(it) for _ in range(ir)]
        rows_v = [next(it) for _ in range(nbuf)]
        acc = next(it)
        if with_cnt:
            ones_v, zcol_v, acc1 = next(it), next(it), next(it)
        isem = [next(it) for _ in range(ir)]
        gsem = [next(it) for _ in range(nbuf)]

        c = lax.axis_index("c")
        s = lax.axis_index("s")
        wid = s * NC + c
        ch0 = wid * nch

        def ixload(i, a):
            return pltpu.make_async_copy(edges_hbm.at[ch0 + i], ix[a], isem[a])

        def gather(b, a):
            return pltpu.make_async_copy(
                rows_hbm.at[ix[a].at[0]], rows_v[b], gsem[b])

        def scatter(b, a):
            pltpu.sync_copy(rows_v[b], acc.at[ix[a].at[1]], add=True)
            if with_cnt:
                pltpu.sync_copy(ones_v, acc1.at[ix[a].at[1]], add=True)

        def zrow(i, carry):
            for j in range(d // 16):
                rows_v[0][i, pl.ds(j * 16, 16)] = jnp.zeros((16,), jnp.float32)
            return carry
        lax.fori_loop(0, K, zrow, 0)
        if with_cnt:
            def zcnt(i, carry):
                ones_v[pl.ds(2 * i, 2), :] = jnp.ones((2, 16), jnp.bfloat16)
                zcol_v[pl.ds(2 * i, 2), :] = jnp.zeros((2, 16), jnp.bfloat16)
                return carry
            lax.fori_loop(0, K // 2, zcnt, 0)

        for a in range(ir):
            ixload(a, a).start()

        def zacc(i, carry):
            t = s + i * NS
            @pl.when(t < nrc)
            def _():
                pltpu.sync_copy(rows_v[0], acc.at[pl.ds(t * K, K)])
                if with_cnt:
                    pltpu.sync_copy(zcol_v, acc1.at[pl.ds(t * K, K)])
            return carry
        lax.fori_loop(0, (nrc + NS - 1) // NS, zacc, 0)
        plsc.subcore_barrier()

        for q in range(nbuf):
            ixload(q, q).wait()
            gather(q, q).start()

        def step(i, q, tail):
            b, a = q % nbuf, q % ir
            gather(b, a).wait()
            scatter(b, a)

            def refill():
                ixload(i + ir, a).start()
            def advance():
                a2 = (q + nbuf) % ir
                ixload(i + nbuf, a2).wait()
                gather(b, a2).start()
            if tail:
                if i + ir < nch:
                    refill()
                if i + nbuf < nch:
                    advance()
            else:
                pl.when(i + ir < nch)(refill)
                pl.when(i + nbuf < nch)(advance)

        def outer(j, carry):
            for q in range(ir):
                step(j * ir + q, q, False)
            return carry
        lax.fori_loop(0, nch // ir, outer, 0)
        for i in range((nch // ir) * ir, nch):
            step(i, i % ir, True)
        plsc.subcore_barrier()

        def cout(i, carry):
            t = s + i * NS
            @pl.when(t < nrc)
            def _():
                pltpu.sync_copy(acc.at[pl.ds(t * K, K)],
                                part_hbm.at[c, pl.ds(t * K, K)])
                if with_cnt:
                    pltpu.sync_copy(acc1.at[pl.ds(t * K, K)],
                                    pcnt_hbm.at[c, pl.ds(t * K, K)])
            return carry
        lax.fori_loop(0, (nrc + NS - 1) // NS, cout, 0)

    return sc_kernel


@functools.lru_cache(maxsize=None)
def _make_tc1(n, f_in, hid, f_out, r):
    grid = n // r

    def body(part, pcnt, x, wl1, bl1, wr1, wl2, wr2, bl2, p, r2, inv):
        a = part[0] + part[1]
        cnt = (pcnt[0, :, 0:1].astype(jnp.float32)
               + pcnt[1, :, 0:1].astype(jnp.float32))
        iv = 1.0 / jnp.maximum(cnt, 1.0)
        mean = a * iv
        h = jnp.maximum(
            jnp.dot(mean, wl1[...], preferred_element_type=jnp.float32)
            + bl1[...]
            + jnp.dot(x[...], wr1[...], preferred_element_type=jnp.float32),
            0.0)
        p[...] = jnp.dot(h, wl2[...], preferred_element_type=jnp.float32)
        r2[...] = (jnp.dot(h, wr2[...], preferred_element_type=jnp.float32)
                   + bl2[...])
        inv[...] = iv

    return pl.pallas_call(
        body,
        grid=(grid,),
        in_specs=[
            pl.BlockSpec((NC, r, f_in), lambda i: (0, i, 0)),
            pl.BlockSpec((NC, r, 16), lambda i: (0, i, 0)),
            pl.BlockSpec((r, f_in), lambda i: (i, 0)),
            pl.BlockSpec((f_in, hid), lambda i: (0, 0)),
            pl.BlockSpec((1, hid), lambda i: (0, 0)),
            pl.BlockSpec((f_in, hid), lambda i: (0, 0)),
            pl.BlockSpec((hid, f_out), lambda i: (0, 0)),
            pl.BlockSpec((hid, f_out), lambda i: (0, 0)),
            pl.BlockSpec((1, f_out), lambda i: (0, 0)),
        ],
        out_specs=[
            pl.BlockSpec((r, f_out), lambda i: (i, 0)),
            pl.BlockSpec((r, f_out), lambda i: (i, 0)),
            pl.BlockSpec((r, 1), lambda i: (i, 0)),
        ],
        out_shape=[
            jax.ShapeDtypeStruct((n, f_out), jnp.float32),
            jax.ShapeDtypeStruct((n, f_out), jnp.float32),
            jax.ShapeDtypeStruct((n, 1), jnp.float32),
        ],
    )


@functools.lru_cache(maxsize=None)
def _make_tc2(n, f_out, r):
    grid = n // r

    def body(part, inv, r2, out):
        out[...] = (part[0] + part[1]) * inv[...] + r2[...]

    return pl.pallas_call(
        body,
        grid=(grid,),
        in_specs=[
            pl.BlockSpec((NC, r, f_out), lambda i: (0, i, 0)),
            pl.BlockSpec((r, 1), lambda i: (i, 0)),
            pl.BlockSpec((r, f_out), lambda i: (i, 0)),
        ],
        out_specs=pl.BlockSpec((r, f_out), lambda i: (i, 0)),
        out_shape=jax.ShapeDtypeStruct((n, f_out), jnp.float32),
    )


def kernel(x, edge_index, W_l1, b_l1, W_r1, W_l2, b_l2, W_r2):
    n, f_in = x.shape
    e = edge_index.shape[1]
    hid = W_l1.shape[1]
    f_out = W_l2.shape[1]

    edges = edge_index.reshape(2, e // K, K).transpose(1, 0, 2)

    part1, pcnt = _make_sc_scatter(n, e, f_in, True)(x, edges)
    p, r2, inv = _make_tc1(n, f_in, hid, f_out, 5000)(
        part1, pcnt, x, W_l1, b_l1.reshape(1, hid), W_r1, W_l2, W_r2,
        b_l2.reshape(1, f_out))
    part2 = _make_sc_scatter(n, e, f_out, False)(p, edges)
    out = _make_tc2(n, f_out, 5000)(part2, inv, r2)
    return out
